# baseline (speedup 1.0000x reference)
import functools

import jax
import jax.numpy as jnp
from jax import lax
from jax.experimental import pallas as pl
from jax.experimental.pallas import tpu as pltpu
from jax.experimental.pallas import tpu_sc as plsc

VOCAB = 1000000
EMBED_DIM = 32
BATCH = 16384
HIST = 50
NUM_CORES = 2
NUM_SUBCORES = 16
NUM_WORKERS = NUM_CORES * NUM_SUBCORES

N_IDS = BATCH * HIST
CHUNK = 512
IDS_PER_WORKER = N_IDS // NUM_WORKERS
PAIRS_PER_WORKER = IDS_PER_WORKER // (2 * CHUNK)


def _build_lookup():
    mesh = plsc.VectorSubcoreMesh(core_axis_name="c", subcore_axis_name="s")

    @functools.partial(
        pl.kernel,
        mesh=mesh,
        out_type=jax.ShapeDtypeStruct((N_IDS, EMBED_DIM), jnp.float32),
        scratch_types=[
            pltpu.VMEM((2, CHUNK), jnp.int32),
            pltpu.VMEM((2, CHUNK, EMBED_DIM), jnp.float32),
            pltpu.SemaphoreType.DMA,
            pltpu.SemaphoreType.DMA,
            pltpu.SemaphoreType.DMA,
            pltpu.SemaphoreType.DMA,
        ],
        compiler_params=pltpu.CompilerParams(use_tc_tiling_on_sc=False),
    )
    def lookup(ids_hbm, tab_hbm, out_hbm, idx_v, rows_v, g0, g1, w0, w1):
        gsem = (g0, g1)
        wsem = (w0, w1)
        wid = lax.axis_index("s") * NUM_CORES + lax.axis_index("c")
        base = wid * IDS_PER_WORKER

        def outer(o, carry):
            gathers = [None, None]
            offs = [None, None]
            for b in range(2):
                off = base + (o * 2 + b) * CHUNK
                offs[b] = off

                @pl.when(o > 0)
                def _(b=b):
                    pltpu.make_async_copy(
                        out_hbm.at[pl.ds(0, CHUNK)], rows_v.at[b], wsem[b]
                    ).wait()

                pltpu.sync_copy(ids_hbm.at[pl.ds(off, CHUNK)], idx_v.at[b])
                gathers[b] = pltpu.async_copy(
                    tab_hbm.at[idx_v.at[b]], rows_v.at[b], gsem[b]
                )
            for b in range(2):
                gathers[b].wait()
                pltpu.async_copy(
                    rows_v.at[b], out_hbm.at[pl.ds(offs[b], CHUNK)], wsem[b]
                )
            return carry

        lax.fori_loop(0, PAIRS_PER_WORKER, outer, 0)
        for b in range(2):
            pltpu.make_async_copy(
                out_hbm.at[pl.ds(0, CHUNK)], rows_v.at[b], wsem[b]
            ).wait()

    return lookup


def kernel(scrim_ids, table):
    ids_flat = scrim_ids.reshape(-1)
    out = _build_lookup()(ids_flat, table)
    return out.reshape(BATCH, HIST, EMBED_DIM)


# speedup vs baseline: 1.7476x; 1.7476x over previous
import functools

import jax
import jax.numpy as jnp
from jax import lax
from jax.experimental import pallas as pl
from jax.experimental.pallas import tpu as pltpu
from jax.experimental.pallas import tpu_sc as plsc

VOCAB = 1000000
EMBED_DIM = 32
BATCH = 16384
HIST = 50
NUM_CORES = 2
NUM_SUBCORES = 16
NUM_WORKERS = NUM_CORES * NUM_SUBCORES

N_IDS = BATCH * HIST
BGROUPS = BATCH // 128
N_UNITS = HIST * BGROUPS
UNITS_PER_WORKER = N_UNITS // NUM_WORKERS


def _build_lookup():
    mesh = plsc.VectorSubcoreMesh(core_axis_name="c", subcore_axis_name="s")

    @functools.partial(
        pl.kernel,
        mesh=mesh,
        out_type=jax.ShapeDtypeStruct((N_IDS * EMBED_DIM,), jnp.float32),
        scratch_types=[
            pltpu.VMEM((2, 128), jnp.int32),
            pltpu.VMEM((2, 128, EMBED_DIM), jnp.float32),
            pltpu.VMEM((2, EMBED_DIM * 128), jnp.float32),
            pltpu.SemaphoreType.DMA,
            pltpu.SemaphoreType.DMA,
            pltpu.SemaphoreType.DMA,
            pltpu.SemaphoreType.DMA,
        ],
        compiler_params=pltpu.CompilerParams(
            use_tc_tiling_on_sc=False, needs_layout_passes=False
        ),
    )
    def lookup(ids_hbm, tab_hbm, out_hbm, idx_v, rows_v, tbuf, g0, g1, w0, w1):
        lane = lax.iota(jnp.int32, 16)
        gsem = (g0, g1)
        wsem = (w0, w1)
        wid = lax.axis_index("s") * NUM_CORES + lax.axis_index("c")
        base_unit = wid * UNITS_PER_WORKER

        def outer(o, carry):
            gathers = [None, None]
            units = [None, None]
            for b in range(2):
                u = base_unit + o * 2 + b
                units[b] = u

                @pl.when(o > 0)
                def _(b=b):
                    pltpu.make_async_copy(
                        out_hbm.at[pl.ds(0, 4096)], tbuf.at[b], wsem[b]
                    ).wait()

                pltpu.sync_copy(ids_hbm.at[pl.ds(u * 128, 128)], idx_v.at[b])
                gathers[b] = pltpu.async_copy(
                    tab_hbm.at[idx_v.at[b]], rows_v.at[b], gsem[b]
                )
            for b in range(2):
                gathers[b].wait()

                @plsc.parallel_loop(0, EMBED_DIM, unroll=4)
                def body(e, b=b):
                    ecol = jnp.full((16,), e, jnp.int32)
                    for g in range(8):
                        vec = plsc.load_gather(
                            rows_v.at[b], [g * 16 + lane, ecol]
                        )
                        tbuf[b, pl.ds(e * 128 + g * 16, 16)] = vec

                h = units[b] // BGROUPS
                bt = units[b] % BGROUPS
                for te in range(4):
                    pltpu.async_copy(
                        tbuf.at[b].at[pl.ds(te * 1024, 1024)],
                        out_hbm.at[pl.ds(((h * 4 + te) * BGROUPS + bt) * 1024, 1024)],
                        wsem[b],
                    )
            return carry

        lax.fori_loop(0, UNITS_PER_WORKER // 2, outer, 0)
        for b in range(2):
            pltpu.make_async_copy(
                out_hbm.at[pl.ds(0, 4096)], tbuf.at[b], wsem[b]
            ).wait()

    return lookup


def kernel(scrim_ids, table):
    ids_hm = scrim_ids.T.reshape(-1)
    out_flat = _build_lookup()(ids_hm, table)
    out5 = out_flat.reshape(HIST, 4, BGROUPS, 8, 128)
    return out5.transpose(2, 4, 0, 1, 3).reshape(BATCH, HIST, EMBED_DIM)
